# baseline (speedup 1.0000x reference)
import jax
import jax.numpy as jnp
from jax import lax
from jax.experimental import pallas as pl
from jax.experimental.pallas import tpu as pltpu
from jax.experimental.pallas import tpu_sc as plsc

VOCAB = 1000
BATCH = 1024
SEQ = 50
NC, NS = 2, 16
NW = NC * NS
B_PER_W = BATCH // NW
NBUF = 2


def _gather_body(table_hbm, idx_hbm, out_hbm, idx_v, bufs, gsems, ssems):
    wid = lax.axis_index("s") * NC + lax.axis_index("c")
    base = wid * B_PER_W

    pltpu.sync_copy(idx_hbm.at[pl.ds(base, B_PER_W)], idx_v)

    def start_gather(j, b):
        pltpu.async_copy(table_hbm.at[idx_v.at[j]], bufs[b], gsems[b])

    def start_store(j, b):
        pltpu.async_copy(bufs[b], out_hbm.at[base + j], ssems[b])

    def wait_gather(b):
        pltpu.make_async_copy(
            table_hbm.at[idx_v.at[0]], bufs[b], gsems[b]
        ).wait()

    def wait_store(b):
        pltpu.make_async_copy(bufs[b], out_hbm.at[base], ssems[b]).wait()

    for b in range(NBUF):
        start_gather(b, b)

    @pl.loop(0, B_PER_W, step=NBUF)
    def _(g):
        for b in range(NBUF):
            j = g + b
            wait_gather(b)
            start_store(j, b)

            @pl.when(j + NBUF < B_PER_W)
            def _():
                wait_store(b)
                start_gather(j + NBUF, b)

    for b in range(NBUF):
        wait_store(b)


@jax.jit
def _lookup(idx, table):
    mesh = plsc.VectorSubcoreMesh(core_axis_name="c", subcore_axis_name="s")
    run = pl.kernel(
        _gather_body,
        out_type=jax.ShapeDtypeStruct((BATCH, SEQ, VOCAB), jnp.float32),
        mesh=mesh,
        compiler_params=pltpu.CompilerParams(use_tc_tiling_on_sc=False),
        scratch_types=[
            pltpu.VMEM((B_PER_W, SEQ), jnp.int32),
            [pltpu.VMEM((SEQ, VOCAB), jnp.float32) for _ in range(NBUF)],
            [pltpu.SemaphoreType.DMA for _ in range(NBUF)],
            [pltpu.SemaphoreType.DMA for _ in range(NBUF)],
        ],
    )
    return run(table, idx)


def kernel(idx, table):
    return _lookup(idx.astype(jnp.int32), table)
